# baseline (speedup 1.0000x reference)
import functools

import jax
import jax.numpy as jnp
from jax import lax
from jax.experimental import pallas as pl
from jax.experimental.pallas import tpu as pltpu
from jax.experimental.pallas import tpu_sc as plsc

DIM = 64
B_TOTAL = 4096 * 200
NC = 2
NS = 16
NW = NC * NS
B_PER_W = B_TOTAL // NW
G = 128
CHUNKS = B_PER_W // G

_mesh = plsc.VectorSubcoreMesh(core_axis_name="c", subcore_axis_name="s")


@functools.partial(
    pl.kernel,
    mesh=_mesh,
    out_type=jax.ShapeDtypeStruct((B_TOTAL, DIM), jnp.float32),
    scratch_types=[
        pltpu.VMEM((G,), jnp.int32),
        pltpu.VMEM((G, DIM), jnp.float32),
        pltpu.SemaphoreType.DMA,
    ],
    compiler_params=pltpu.CompilerParams(use_tc_tiling_on_sc=False),
)
def _embed_gather(x_hbm, w_hbm, out_hbm, idx_v, rows_v, sem):
    wid = lax.axis_index("s") * NC + lax.axis_index("c")
    base = wid * B_PER_W

    def body(j, carry):
        cb = base + j * G
        pltpu.sync_copy(x_hbm.at[pl.ds(cb, G)], idx_v)
        pltpu.async_copy(w_hbm.at[idx_v], rows_v, sem).wait()
        pltpu.sync_copy(rows_v, out_hbm.at[pl.ds(cb, G)])
        return carry

    lax.fori_loop(0, CHUNKS, body, 0)


def kernel(x, embed_weight):
    flat = x.reshape(-1)
    out = _embed_gather(flat, embed_weight)
    return out.reshape(x.shape + (DIM,))
